# Initial kernel scaffold; baseline (speedup 1.0000x reference)
import jax
import jax.numpy as jnp
from jax.experimental import pallas as pl


def kernel(story, query, emb, T):
    raise NotImplementedError("write your pallas kernel here")


# speedup vs baseline: 6.9520x; 6.9520x over previous
import functools

import jax
import jax.numpy as jnp
from jax import lax
from jax.experimental import pallas as pl
from jax.experimental.pallas import tpu as pltpu
from jax.experimental.pallas import tpu_sc as plsc

D = 64
HOPS = 3
V = 100000
S = 50
B = 1024
SENT = 20
QLEN = 20

NC, NS, L = 2, 16, 16
NW = NC * NS
B_PER_W = B // NW
WORDS = S * SENT
WORDS_PAD = 1008
NVEC = WORDS_PAD // L
QW = B_PER_W * QLEN


def _pool_body(story_hbm, query_hbm, emb_hbm, pooled_hbm, u0_hbm,
               idx_v, idxk_v, rows_v, pool_v, u0_v, sem):
  cid = lax.axis_index("c")
  sid = lax.axis_index("s")
  wid = sid * NC + cid
  b0 = wid * B_PER_W

  pltpu.sync_copy(query_hbm.at[pl.ds(wid * QW, QW)], idx_v.at[pl.ds(0, QW)])
  pltpu.async_copy(emb_hbm.at[idx_v.at[pl.ds(0, QW)]],
                   rows_v.at[pl.ds(0, QW)], sem).wait()

  def q_body(bi, _):
    base = bi * QLEN
    for c in range(D // L):
      acc = rows_v[base, pl.ds(c * L, L)]
      for j in range(1, QLEN):
        acc = acc + rows_v[base + j, pl.ds(c * L, L)]
      u0_v[bi, pl.ds(c * L, L)] = acc
    return _
  lax.fori_loop(0, B_PER_W, q_body, None)
  pltpu.sync_copy(u0_v, u0_hbm.at[pl.ds(b0, B_PER_W)])

  def batch_body(bi, _):
    b = b0 + bi
    pltpu.sync_copy(story_hbm.at[b], idx_v)
    for k in range(HOPS + 1):
      if k == 0:
        gidx = idx_v
      else:
        off = jnp.int32(k * V)
        def addoff(i, c):
          idxk_v[pl.ds(i * L, L)] = idx_v[pl.ds(i * L, L)] + off
          return c
        lax.fori_loop(0, NVEC, addoff, None)
        gidx = idxk_v
      pltpu.async_copy(emb_hbm.at[gidx], rows_v, sem).wait()

      def sent_body(s, c):
        base = s * SENT
        for cc in range(D // L):
          acc = rows_v[base, pl.ds(cc * L, L)]
          for j in range(1, SENT):
            acc = acc + rows_v[base + j, pl.ds(cc * L, L)]
          pool_v[s, pl.ds(cc * L, L)] = acc
        return c
      lax.fori_loop(0, S, sent_body, None)
      pltpu.sync_copy(pool_v, pooled_hbm.at[k, b])
    return _
  lax.fori_loop(0, B_PER_W, batch_body, None)


@jax.jit
def _pool_call(story_idx, query_flat, emb_flat):
  mesh = plsc.VectorSubcoreMesh(core_axis_name="c", subcore_axis_name="s",
                                num_cores=NC, num_subcores=NS)
  return pl.kernel(
      _pool_body,
      out_type=(jax.ShapeDtypeStruct((HOPS + 1, B, S, D), jnp.float32),
                jax.ShapeDtypeStruct((B, D), jnp.float32)),
      mesh=mesh,
      compiler_params=pltpu.CompilerParams(use_tc_tiling_on_sc=False),
      scratch_types=[
          pltpu.VMEM((WORDS_PAD,), jnp.int32),
          pltpu.VMEM((WORDS_PAD,), jnp.int32),
          pltpu.VMEM((WORDS_PAD, D), jnp.float32),
          pltpu.VMEM((S, D), jnp.float32),
          pltpu.VMEM((B_PER_W, D), jnp.float32),
          pltpu.SemaphoreType.DMA,
      ],
  )(story_idx, query_flat, emb_flat)


BT = 64


def _hops_body(pooled_ref, t_ref, u0_ref, u_ref):
  u = u0_ref[...]
  for hop in range(HOPS):
    m = pooled_ref[hop] + t_ref[hop][None]
    logits = jnp.sum(m * u[:, None, :], axis=2)
    logits = logits - jnp.max(logits, axis=1, keepdims=True)
    e = jnp.exp(logits)
    p = e / jnp.sum(e, axis=1, keepdims=True)
    c = pooled_ref[hop + 1] + t_ref[hop + 1][None]
    o = jnp.sum(p[:, :, None] * c, axis=1)
    u = u + o
  u_ref[...] = u


@jax.jit
def _hops_call(pooled, t, u0):
  return pl.pallas_call(
      _hops_body,
      grid=(B // BT,),
      in_specs=[
          pl.BlockSpec((HOPS + 1, BT, S, D), lambda i: (0, i, 0, 0)),
          pl.BlockSpec((HOPS + 1, S, D), lambda i: (0, 0, 0)),
          pl.BlockSpec((BT, D), lambda i: (i, 0)),
      ],
      out_specs=pl.BlockSpec((BT, D), lambda i: (i, 0)),
      out_shape=jax.ShapeDtypeStruct((B, D), jnp.float32),
  )(pooled, t, u0)


VT = 1024
NFULL = V // VT
TAIL = V - NFULL * VT
NVT = NFULL + 1


def _tile_logits(u_ref, w_ref):
  return lax.dot_general(u_ref[...], w_ref[0], (((1,), (1,)), ((), ())),
                         preferred_element_type=jnp.float32)


def _stats_body(u_ref, w_ref, wt_ref, mx_ref, l_ref, m_s, l_s):
  j = pl.program_id(0)

  def update(s):
    bm = jnp.max(s, axis=1, keepdims=True)

    @pl.when(j == 0)
    def _():
      m_s[...] = bm
      l_s[...] = jnp.sum(jnp.exp(s - bm), axis=1, keepdims=True)

    @pl.when(j > 0)
    def _():
      m_old = m_s[...]
      m_new = jnp.maximum(m_old, bm)
      l_s[...] = (l_s[...] * jnp.exp(m_old - m_new)
                  + jnp.sum(jnp.exp(s - m_new), axis=1, keepdims=True))
      m_s[...] = m_new

  @pl.when(j < NFULL)
  def _():
    update(_tile_logits(u_ref, w_ref))

  @pl.when(j == NFULL)
  def _():
    st = lax.dot_general(u_ref[...], wt_ref[...], (((1,), (1,)), ((), ())),
                         preferred_element_type=jnp.float32)
    update(st)
    mx_ref[...] = m_s[...]
    l_ref[...] = l_s[...]


@jax.jit
def _stats_call(u, emb, w_tail):
  return pl.pallas_call(
      _stats_body,
      grid=(NVT,),
      in_specs=[
          pl.BlockSpec((B, D), lambda j: (0, 0)),
          pl.BlockSpec((1, VT, D), lambda j: (HOPS, jnp.minimum(j, NFULL - 1), 0)),
          pl.BlockSpec((TAIL, D), lambda j: (0, 0)),
      ],
      out_specs=[
          pl.BlockSpec((B, 1), lambda j: (0, 0)),
          pl.BlockSpec((B, 1), lambda j: (0, 0)),
      ],
      out_shape=[jax.ShapeDtypeStruct((B, 1), jnp.float32),
                 jax.ShapeDtypeStruct((B, 1), jnp.float32)],
      scratch_shapes=[pltpu.VMEM((B, 1), jnp.float32),
                      pltpu.VMEM((B, 1), jnp.float32)],
  )(u, emb, w_tail)


def _out_body(u_ref, w_ref, wt_ref, mx_ref, l_ref, ahat_hbm, soft_hbm,
              sa, ss, sat, sst, sem_a, sem_s, sem_t):
  j = pl.program_id(0)
  mx = mx_ref[...]
  linv = 1.0 / l_ref[...]

  @pl.when(j < NFULL)
  def _():
    s = _tile_logits(u_ref, w_ref)
    soft = jnp.exp(s - mx) * linv
    for par in range(2):
      @pl.when(j % 2 == par)
      def _():
        @pl.when(j >= 2)
        def _():
          pltpu.make_async_copy(
              sa.at[par], ahat_hbm.at[:, pl.ds((j - 2) * VT, VT)],
              sem_a.at[par]).wait()
          pltpu.make_async_copy(
              ss.at[par], soft_hbm.at[:, pl.ds((j - 2) * VT, VT)],
              sem_s.at[par]).wait()
        sa[par] = s
        ss[par] = soft
        off = pl.multiple_of(j * VT, 128)
        pltpu.make_async_copy(
            sa.at[par], ahat_hbm.at[:, pl.ds(off, VT)], sem_a.at[par]).start()
        pltpu.make_async_copy(
            ss.at[par], soft_hbm.at[:, pl.ds(off, VT)], sem_s.at[par]).start()

  @pl.when(j == NFULL)
  def _():
    st = lax.dot_general(u_ref[...], wt_ref[...], (((1,), (1,)), ((), ())),
                         preferred_element_type=jnp.float32)
    sat[...] = st
    sst[...] = jnp.exp(st - mx) * linv
    pltpu.make_async_copy(
        sat, ahat_hbm.at[:, pl.ds(NFULL * VT, TAIL)], sem_t).start()
    pltpu.make_async_copy(
        sst, soft_hbm.at[:, pl.ds(NFULL * VT, TAIL)], sem_t).start()
    for jj in (j - 2, j - 1):
      par = jj % 2
      pltpu.make_async_copy(
          sa.at[par], ahat_hbm.at[:, pl.ds(jj * VT, VT)], sem_a.at[par]).wait()
      pltpu.make_async_copy(
          ss.at[par], soft_hbm.at[:, pl.ds(jj * VT, VT)], sem_s.at[par]).wait()
    pltpu.make_async_copy(
        sat, ahat_hbm.at[:, pl.ds(NFULL * VT, TAIL)], sem_t).wait()
    pltpu.make_async_copy(
        sst, soft_hbm.at[:, pl.ds(NFULL * VT, TAIL)], sem_t).wait()


@jax.jit
def _out_call(u, emb, w_tail, mx, l):
  return pl.pallas_call(
      _out_body,
      grid=(NVT,),
      in_specs=[
          pl.BlockSpec((B, D), lambda j: (0, 0)),
          pl.BlockSpec((1, VT, D), lambda j: (HOPS, jnp.minimum(j, NFULL - 1), 0)),
          pl.BlockSpec((TAIL, D), lambda j: (0, 0)),
          pl.BlockSpec((B, 1), lambda j: (0, 0)),
          pl.BlockSpec((B, 1), lambda j: (0, 0)),
      ],
      out_specs=[
          pl.BlockSpec(memory_space=pl.ANY),
          pl.BlockSpec(memory_space=pl.ANY),
      ],
      out_shape=[jax.ShapeDtypeStruct((B, V), jnp.float32),
                 jax.ShapeDtypeStruct((B, V), jnp.float32)],
      scratch_shapes=[
          pltpu.VMEM((2, B, VT), jnp.float32),
          pltpu.VMEM((2, B, VT), jnp.float32),
          pltpu.VMEM((B, TAIL), jnp.float32),
          pltpu.VMEM((B, TAIL), jnp.float32),
          pltpu.SemaphoreType.DMA((2,)),
          pltpu.SemaphoreType.DMA((2,)),
          pltpu.SemaphoreType.DMA,
      ],
  )(u, emb, w_tail, mx, l)


def kernel(story, query, emb, T):
  story_idx = jnp.pad(story.reshape(B, WORDS).astype(jnp.int32),
                      ((0, 0), (0, WORDS_PAD - WORDS)))
  query_flat = query.reshape(-1).astype(jnp.int32)
  emb_flat = emb.reshape((HOPS + 1) * V, D)
  w_tail = lax.slice(emb, (HOPS, NFULL * VT, 0), (HOPS + 1, V, D)).reshape(TAIL, D)
  pooled, u0 = _pool_call(story_idx, query_flat, emb_flat)
  u = _hops_call(pooled, T, u0)
  mx, l = _stats_call(u, emb, w_tail)
  ahat, soft = _out_call(u, emb, w_tail, mx, l)
  return ahat, soft


# speedup vs baseline: 7.2921x; 1.0489x over previous
import functools

import jax
import jax.numpy as jnp
from jax import lax
from jax.experimental import pallas as pl
from jax.experimental.pallas import tpu as pltpu
from jax.experimental.pallas import tpu_sc as plsc

D = 64
HOPS = 3
V = 100000
S = 50
B = 1024
SENT = 20
QLEN = 20

NC, NS, L = 2, 16, 16
NW = NC * NS
B_PER_W = B // NW
HALF = 500
HALF_PAD = 512
SENT_PER_HALF = HALF // SENT
QW = B_PER_W * QLEN
QHB = B_PER_W // 2
UNITS = B_PER_W * (HOPS + 1) * 2


def _pool_body(story_hbm, query_hbm, emb_hbm, pooled_hbm, u0_hbm,
               idx_v, idxk_v, rows_v, pool_v, u0_v, sems):
  cid = lax.axis_index("c")
  sid = lax.axis_index("s")
  wid = sid * NC + cid
  b0 = wid * B_PER_W

  for h in range(2):
    pltpu.sync_copy(query_hbm.at[pl.ds(wid * QW + h * (QW // 2), QW // 2)],
                    idx_v.at[0, pl.ds(0, QW // 2)])
    pltpu.async_copy(emb_hbm.at[idx_v.at[0, pl.ds(0, QW // 2)]],
                     rows_v.at[0, pl.ds(0, QW // 2)], sems.at[0]).wait()

    def q_body(bi, _):
      base = bi * QLEN
      for c in range(D // L):
        acc = rows_v[0, base, pl.ds(c * L, L)]
        for j in range(1, QLEN):
          acc = acc + rows_v[0, base + j, pl.ds(c * L, L)]
        u0_v[bi, pl.ds(c * L, L)] = acc
      return _
    lax.fori_loop(0, QHB, q_body, None)
    pltpu.sync_copy(u0_v.at[pl.ds(0, QHB)],
                    u0_hbm.at[pl.ds(b0 + h * QHB, QHB)])

  def unit_body(n, _):
    par = lax.rem(n, 2)
    h = lax.rem(n, 2)
    k = lax.rem(n // 2, HOPS + 1)
    bi = n // (2 * (HOPS + 1))

    @pl.when(n < UNITS)
    def _():
      for p in range(2):
        @pl.when(par == p)
        def _():
          pltpu.sync_copy(story_hbm.at[b0 + bi, h], idx_v.at[p])
          off = (k * V).astype(jnp.int32)
          def addoff(i, c):
            idxk_v[p, pl.ds(i * L, L)] = idx_v[p, pl.ds(i * L, L)] + off
            return c
          lax.fori_loop(0, HALF_PAD // L, addoff, None)
          pltpu.make_async_copy(emb_hbm.at[idxk_v.at[p]], rows_v.at[p],
                                sems.at[p]).start()

    @pl.when(n > 0)
    def _():
      m = n - 1
      mpar = lax.rem(m, 2)
      mh = lax.rem(m, 2)
      mk = lax.rem(m // 2, HOPS + 1)
      mbi = m // (2 * (HOPS + 1))
      for p in range(2):
        @pl.when(mpar == p)
        def _():
          pltpu.make_async_copy(emb_hbm.at[idxk_v.at[p]], rows_v.at[p],
                                sems.at[p]).wait()

          def sent_body(s, c):
            base = s * SENT
            for cc in range(D // L):
              acc = rows_v[p, base, pl.ds(cc * L, L)]
              for j in range(1, SENT):
                acc = acc + rows_v[p, base + j, pl.ds(cc * L, L)]
              pool_v[s, pl.ds(cc * L, L)] = acc
            return c
          lax.fori_loop(0, SENT_PER_HALF, sent_body, None)
          pltpu.sync_copy(
              pool_v,
              pooled_hbm.at[mk, b0 + mbi,
                            pl.ds(mh * SENT_PER_HALF, SENT_PER_HALF)])
    return _

  lax.fori_loop(0, UNITS + 1, unit_body, None)


@jax.jit
def _pool_call(story_idx, query_flat, emb_flat):
  mesh = plsc.VectorSubcoreMesh(core_axis_name="c", subcore_axis_name="s",
                                num_cores=NC, num_subcores=NS)
  return pl.kernel(
      _pool_body,
      out_type=(jax.ShapeDtypeStruct((HOPS + 1, B, S, D), jnp.float32),
                jax.ShapeDtypeStruct((B, D), jnp.float32)),
      mesh=mesh,
      compiler_params=pltpu.CompilerParams(use_tc_tiling_on_sc=False),
      scratch_types=[
          pltpu.VMEM((2, HALF_PAD), jnp.int32),
          pltpu.VMEM((2, HALF_PAD), jnp.int32),
          pltpu.VMEM((2, HALF_PAD, D), jnp.float32),
          pltpu.VMEM((SENT_PER_HALF, D), jnp.float32),
          pltpu.VMEM((B_PER_W, D), jnp.float32),
          pltpu.SemaphoreType.DMA((2,)),
      ],
  )(story_idx, query_flat, emb_flat)


BT = 64


def _hops_body(pooled_ref, t_ref, u0_ref, u_ref):
  u = u0_ref[...]
  for hop in range(HOPS):
    m = pooled_ref[hop] + t_ref[hop][None]
    logits = jnp.sum(m * u[:, None, :], axis=2)
    logits = logits - jnp.max(logits, axis=1, keepdims=True)
    e = jnp.exp(logits)
    p = e / jnp.sum(e, axis=1, keepdims=True)
    c = pooled_ref[hop + 1] + t_ref[hop + 1][None]
    o = jnp.sum(p[:, :, None] * c, axis=1)
    u = u + o
  u_ref[...] = u


@jax.jit
def _hops_call(pooled, t, u0):
  return pl.pallas_call(
      _hops_body,
      grid=(B // BT,),
      in_specs=[
          pl.BlockSpec((HOPS + 1, BT, S, D), lambda i: (0, i, 0, 0)),
          pl.BlockSpec((HOPS + 1, S, D), lambda i: (0, 0, 0)),
          pl.BlockSpec((BT, D), lambda i: (i, 0)),
      ],
      out_specs=pl.BlockSpec((BT, D), lambda i: (i, 0)),
      out_shape=jax.ShapeDtypeStruct((B, D), jnp.float32),
  )(pooled, t, u0)


VT = 1024
NFULL = V // VT
TAIL = V - NFULL * VT
NVT = NFULL + 1


def _tile_logits(u_ref, w_ref):
  return lax.dot_general(u_ref[...], w_ref[0], (((1,), (1,)), ((), ())),
                         preferred_element_type=jnp.float32)


def _stats_body(u_ref, w_ref, wt_ref, mx_ref, l_ref, m_s, l_s):
  j = pl.program_id(0)

  def update(s):
    bm = jnp.max(s, axis=1, keepdims=True)

    @pl.when(j == 0)
    def _():
      m_s[...] = bm
      l_s[...] = jnp.sum(jnp.exp(s - bm), axis=1, keepdims=True)

    @pl.when(j > 0)
    def _():
      m_old = m_s[...]
      m_new = jnp.maximum(m_old, bm)
      l_s[...] = (l_s[...] * jnp.exp(m_old - m_new)
                  + jnp.sum(jnp.exp(s - m_new), axis=1, keepdims=True))
      m_s[...] = m_new

  @pl.when(j < NFULL)
  def _():
    update(_tile_logits(u_ref, w_ref))

  @pl.when(j == NFULL)
  def _():
    st = lax.dot_general(u_ref[...], wt_ref[...], (((1,), (1,)), ((), ())),
                         preferred_element_type=jnp.float32)
    update(st)
    mx_ref[...] = m_s[...]
    l_ref[...] = l_s[...]


@jax.jit
def _stats_call(u, emb, w_tail):
  return pl.pallas_call(
      _stats_body,
      grid=(NVT,),
      in_specs=[
          pl.BlockSpec((B, D), lambda j: (0, 0)),
          pl.BlockSpec((1, VT, D), lambda j: (HOPS, jnp.minimum(j, NFULL - 1), 0)),
          pl.BlockSpec((TAIL, D), lambda j: (0, 0)),
      ],
      out_specs=[
          pl.BlockSpec((B, 1), lambda j: (0, 0)),
          pl.BlockSpec((B, 1), lambda j: (0, 0)),
      ],
      out_shape=[jax.ShapeDtypeStruct((B, 1), jnp.float32),
                 jax.ShapeDtypeStruct((B, 1), jnp.float32)],
      scratch_shapes=[pltpu.VMEM((B, 1), jnp.float32),
                      pltpu.VMEM((B, 1), jnp.float32)],
  )(u, emb, w_tail)


def _out_body(u_ref, w_ref, wt_ref, mx_ref, l_ref, ahat_hbm, soft_hbm,
              sa, ss, sat, sst, sem_a, sem_s, sem_t):
  j = pl.program_id(0)
  mx = mx_ref[...]
  linv = 1.0 / l_ref[...]

  @pl.when(j < NFULL)
  def _():
    s = _tile_logits(u_ref, w_ref)
    soft = jnp.exp(s - mx) * linv
    for par in range(2):
      @pl.when(j % 2 == par)
      def _():
        @pl.when(j >= 2)
        def _():
          pltpu.make_async_copy(
              sa.at[par], ahat_hbm.at[:, pl.ds((j - 2) * VT, VT)],
              sem_a.at[par]).wait()
          pltpu.make_async_copy(
              ss.at[par], soft_hbm.at[:, pl.ds((j - 2) * VT, VT)],
              sem_s.at[par]).wait()
        sa[par] = s
        ss[par] = soft
        off = pl.multiple_of(j * VT, 128)
        pltpu.make_async_copy(
            sa.at[par], ahat_hbm.at[:, pl.ds(off, VT)], sem_a.at[par]).start()
        pltpu.make_async_copy(
            ss.at[par], soft_hbm.at[:, pl.ds(off, VT)], sem_s.at[par]).start()

  @pl.when(j == NFULL)
  def _():
    st = lax.dot_general(u_ref[...], wt_ref[...], (((1,), (1,)), ((), ())),
                         preferred_element_type=jnp.float32)
    sat[...] = st
    sst[...] = jnp.exp(st - mx) * linv
    pltpu.make_async_copy(
        sat, ahat_hbm.at[:, pl.ds(NFULL * VT, TAIL)], sem_t).start()
    pltpu.make_async_copy(
        sst, soft_hbm.at[:, pl.ds(NFULL * VT, TAIL)], sem_t).start()
    for jj in (j - 2, j - 1):
      par = jj % 2
      pltpu.make_async_copy(
          sa.at[par], ahat_hbm.at[:, pl.ds(jj * VT, VT)], sem_a.at[par]).wait()
      pltpu.make_async_copy(
          ss.at[par], soft_hbm.at[:, pl.ds(jj * VT, VT)], sem_s.at[par]).wait()
    pltpu.make_async_copy(
        sat, ahat_hbm.at[:, pl.ds(NFULL * VT, TAIL)], sem_t).wait()
    pltpu.make_async_copy(
        sst, soft_hbm.at[:, pl.ds(NFULL * VT, TAIL)], sem_t).wait()


@jax.jit
def _out_call(u, emb, w_tail, mx, l):
  return pl.pallas_call(
      _out_body,
      grid=(NVT,),
      in_specs=[
          pl.BlockSpec((B, D), lambda j: (0, 0)),
          pl.BlockSpec((1, VT, D), lambda j: (HOPS, jnp.minimum(j, NFULL - 1), 0)),
          pl.BlockSpec((TAIL, D), lambda j: (0, 0)),
          pl.BlockSpec((B, 1), lambda j: (0, 0)),
          pl.BlockSpec((B, 1), lambda j: (0, 0)),
      ],
      out_specs=[
          pl.BlockSpec(memory_space=pl.ANY),
          pl.BlockSpec(memory_space=pl.ANY),
      ],
      out_shape=[jax.ShapeDtypeStruct((B, V), jnp.float32),
                 jax.ShapeDtypeStruct((B, V), jnp.float32)],
      scratch_shapes=[
          pltpu.VMEM((2, B, VT), jnp.float32),
          pltpu.VMEM((2, B, VT), jnp.float32),
          pltpu.VMEM((B, TAIL), jnp.float32),
          pltpu.VMEM((B, TAIL), jnp.float32),
          pltpu.SemaphoreType.DMA((2,)),
          pltpu.SemaphoreType.DMA((2,)),
          pltpu.SemaphoreType.DMA,
      ],
  )(u, emb, w_tail, mx, l)


def kernel(story, query, emb, T):
  flat = story.reshape(B, 2, HALF).astype(jnp.int32)
  story_idx = jnp.pad(flat, ((0, 0), (0, 0), (0, HALF_PAD - HALF)))
  query_flat = query.reshape(-1).astype(jnp.int32)
  emb_flat = emb.reshape((HOPS + 1) * V, D)
  w_tail = lax.slice(emb, (HOPS, NFULL * VT, 0), (HOPS + 1, V, D)).reshape(TAIL, D)
  pooled, u0 = _pool_call(story_idx, query_flat, emb_flat)
  u = _hops_call(pooled, T, u0)
  mx, l = _stats_call(u, emb, w_tail)
  ahat, soft = _out_call(u, emb, w_tail, mx, l)
  return ahat, soft
